# baseline (speedup 1.0000x reference)
import jax
import jax.numpy as jnp
from jax import lax
from jax.experimental import pallas as pl
from jax.experimental.pallas import tpu as pltpu
from jax.experimental.pallas import tpu_sc as plsc

BATCH = 16384
VOCAB = 1000000
EMB = 32
H0 = 128
H1 = 64
PACK = 128 // EMB
VROWS = VOCAB // PACK

_NC = 2
_NS = 16
_NW = _NC * _NS
_BPW = BATCH // _NW
_SUB = _BPW // 2


def _sc_gather_body(q0_hbm, q1_hbm, e0_hbm, e1_hbm, out0_hbm, out1_hbm,
                    i0a, i0b, i1a, i1b, bufa, bufb, sema, semb):
    wid = lax.axis_index("s") * _NC + lax.axis_index("c")
    base = wid * _BPW
    pltpu.sync_copy(q0_hbm.at[pl.ds(base, _SUB)], i0a)
    pltpu.sync_copy(q0_hbm.at[pl.ds(base + _SUB, _SUB)], i0b)
    pltpu.sync_copy(q1_hbm.at[pl.ds(base, _SUB)], i1a)
    pltpu.sync_copy(q1_hbm.at[pl.ds(base + _SUB, _SUB)], i1b)
    cpa = pltpu.async_copy(e0_hbm.at[i0a], bufa, sema)
    cpb = pltpu.async_copy(e0_hbm.at[i0b], bufb, semb)
    cpa.wait()
    pltpu.sync_copy(bufa, out0_hbm.at[pl.ds(base, _SUB)])
    cpa2 = pltpu.async_copy(e1_hbm.at[i1a], bufa, sema)
    cpb.wait()
    pltpu.sync_copy(bufb, out0_hbm.at[pl.ds(base + _SUB, _SUB)])
    cpb2 = pltpu.async_copy(e1_hbm.at[i1b], bufb, semb)
    cpa2.wait()
    pltpu.sync_copy(bufa, out1_hbm.at[pl.ds(base, _SUB)])
    cpb2.wait()
    pltpu.sync_copy(bufb, out1_hbm.at[pl.ds(base + _SUB, _SUB)])


_sc_gather = pl.kernel(
    _sc_gather_body,
    out_type=(
        jax.ShapeDtypeStruct((BATCH, 128), jnp.float32),
        jax.ShapeDtypeStruct((BATCH, 128), jnp.float32),
    ),
    mesh=plsc.VectorSubcoreMesh(core_axis_name="c", subcore_axis_name="s"),
    scratch_types=[
        pltpu.VMEM((_SUB,), jnp.int32),
        pltpu.VMEM((_SUB,), jnp.int32),
        pltpu.VMEM((_SUB,), jnp.int32),
        pltpu.VMEM((_SUB,), jnp.int32),
        pltpu.VMEM((_SUB, 128), jnp.float32),
        pltpu.VMEM((_SUB, 128), jnp.float32),
        pltpu.SemaphoreType.DMA,
        pltpu.SemaphoreType.DMA,
    ],
)


_BB = 2048


def _mlp_body(e0_ref, e1_ref, off0_ref, off1_ref, w0a_ref, w0b_ref, b0_ref,
              w1_ref, b1_ref, w2_ref, b2_ref, out_ref):
    grp = lax.broadcasted_iota(jnp.int32, (_BB, 128), 1) // EMB
    m0 = (grp == off0_ref[...]).astype(jnp.float32)
    m1 = (grp == off1_ref[...]).astype(jnp.float32)
    h = jnp.dot(e0_ref[...] * m0, w0a_ref[...],
                preferred_element_type=jnp.float32)
    h += jnp.dot(e1_ref[...] * m1, w0b_ref[...],
                 preferred_element_type=jnp.float32)
    h = jnp.maximum(h + b0_ref[...], 0.0)
    h = jnp.dot(h, w1_ref[...], preferred_element_type=jnp.float32)
    h = jnp.maximum(h + b1_ref[...], 0.0)
    out_ref[...] = jnp.dot(h, w2_ref[...],
                           preferred_element_type=jnp.float32) + b2_ref[...]


@jax.jit
def kernel(x, E0, E1, W0, b0, W1, b1, W2, b2):
    x0 = x[:, 0].astype(jnp.int32)
    x1 = x[:, 1].astype(jnp.int32)
    q0 = x0 >> 2
    q1 = x1 >> 2
    off0 = (x0 & 3).reshape(BATCH, 1)
    off1 = (x1 & 3).reshape(BATCH, 1)
    e0w, e1w = _sc_gather(q0, q1, E0.reshape(VROWS, 128),
                          E1.reshape(VROWS, 128))

    w0a4 = jnp.tile(W0[:, :EMB].T, (PACK, 1))
    w0b4 = jnp.tile(W0[:, EMB:].T, (PACK, 1))
    w1t = W1.T
    w2t = W2.T

    grid = BATCH // _BB
    out = pl.pallas_call(
        _mlp_body,
        grid=(grid,),
        in_specs=[
            pl.BlockSpec((_BB, 128), lambda i: (i, 0)),
            pl.BlockSpec((_BB, 128), lambda i: (i, 0)),
            pl.BlockSpec((_BB, 1), lambda i: (i, 0)),
            pl.BlockSpec((_BB, 1), lambda i: (i, 0)),
            pl.BlockSpec((128, H0), lambda i: (0, 0)),
            pl.BlockSpec((128, H0), lambda i: (0, 0)),
            pl.BlockSpec((1, H0), lambda i: (0, 0)),
            pl.BlockSpec((H0, H1), lambda i: (0, 0)),
            pl.BlockSpec((1, H1), lambda i: (0, 0)),
            pl.BlockSpec((H1, 1), lambda i: (0, 0)),
            pl.BlockSpec((1, 1), lambda i: (0, 0)),
        ],
        out_specs=pl.BlockSpec((_BB, 1), lambda i: (i, 0)),
        out_shape=jax.ShapeDtypeStruct((BATCH, 1), jnp.float32),
    )(e0w, e1w, off0, off1, w0a4, w0b4, b0.reshape(1, H0), w1t,
      b1.reshape(1, H1), w2t, b2.reshape(1, 1))
    return out
